# Initial kernel scaffold; baseline (speedup 1.0000x reference)
import jax
import jax.numpy as jnp
from jax.experimental import pallas as pl


def kernel(h, adj, W, a):
    raise NotImplementedError("write your pallas kernel here")


# speedup vs baseline: 5.4061x; 5.4061x over previous
import functools

import jax
import jax.numpy as jnp
from jax import lax
from jax.experimental import pallas as pl
from jax.experimental.pallas import tpu as pltpu
from jax.experimental.pallas import tpu_sc as plsc

IN_F = 128
OUT_F = 128
ALPHA = 0.2
N = 10000
E = 320000

NC = 2
NS = 16
L = 16
NW = NC * NS
E_PER_W = E // NW
CHUNK = 80
PAD = 16
CHUNK_P = CHUNK + PAD
SB = 25
NSB = E_PER_W // (SB * CHUNK)
N_PAD = 10240
ROWS_PER_TILE = N_PAD // NS
GROUPS = CHUNK // L


def _leaky(x):
    return jnp.maximum(x, ALPHA * x)


def _pre_body(h_ref, w_ref, a1_ref, a2_ref, wh_ref, wh1_ref, wh2_ref, m1_ref):
    wh = jnp.dot(h_ref[...], w_ref[...], preferred_element_type=jnp.float32)
    wh_ref[...] = wh
    wh1 = jnp.dot(wh, a1_ref[...], preferred_element_type=jnp.float32)
    wh2 = jnp.dot(wh, a2_ref[...], preferred_element_type=jnp.float32)
    wh1_ref[...] = wh1
    wh2_ref[...] = wh2
    m1_ref[...] = jnp.max(wh1, keepdims=True)


_pre = pl.pallas_call(
    _pre_body,
    out_shape=[
        jax.ShapeDtypeStruct((N, OUT_F), jnp.float32),
        jax.ShapeDtypeStruct((N, 1), jnp.float32),
        jax.ShapeDtypeStruct((N, 1), jnp.float32),
        jax.ShapeDtypeStruct((1, 1), jnp.float32),
    ],
)


def _sc_body(wh_hbm, wh1_hbm, wh2_hbm, m1_hbm, row_hbm, col_hbm,
             hp_hbm, s_hbm,
             m1_v, ridx_v, cidx_v, w1_v, w2_v, s_v, p_v, rows_v,
             wh1_sh, wh2_sh, hp_sh, sem):
    cid = lax.axis_index("c")
    sid = lax.axis_index("s")
    wid = sid * NC + cid

    pltpu.sync_copy(m1_hbm, m1_v)

    @pl.when(sid == 0)
    def _():
        pltpu.sync_copy(wh1_hbm, wh1_sh)
        pltpu.sync_copy(wh2_hbm, wh2_sh)

    zero16 = jnp.zeros((L,), jnp.float32)

    def _zs(i, _):
        s_v[0, pl.ds(i * L, L)] = zero16
        return 0
    lax.fori_loop(0, N // L, _zs, 0)

    def _zr(i, _):
        for v in range(OUT_F // L):
            rows_v[i, pl.ds(v * L, L)] = zero16
        return 0
    lax.fori_loop(0, CHUNK_P, _zr, 0)

    for g in range(PAD // L):
        p_v[pl.ds(g * L, L)] = zero16

    base = sid * ROWS_PER_TILE
    for k in range(ROWS_PER_TILE // CHUNK):
        pltpu.sync_copy(rows_v.at[pl.ds(0, CHUNK)],
                        hp_sh.at[pl.ds(base + k * CHUNK, CHUNK)])

    plsc.subcore_barrier()

    m1s = m1_v[...]

    def _chunk(c, _):
        idx = ridx_v.at[c, 0]
        cdx = cidx_v.at[c, 0]
        pltpu.sync_copy(wh1_sh.at[idx], w1_v)
        pltpu.sync_copy(wh2_sh.at[cdx], w2_v)
        pltpu.sync_copy(wh_hbm.at[idx], rows_v)

        for g in range(GROUPS):
            sl = pl.ds(PAD + g * L, L)
            w1 = w1_v[sl]
            w2 = w2_v[sl]
            c16 = cidx_v[c, 0, sl]
            e = _leaky(w1 + w2)
            mc = _leaky(w2 + m1s)
            p16 = jnp.exp(e - mc)
            p_v[sl] = p16
            plsc.addupdate_scatter(s_v, [jnp.zeros((L,), jnp.int32), c16], p16)

        for ei in range(CHUNK_P):
            ps = plsc.load_gather(p_v, [jnp.full((L,), ei, jnp.int32)])
            for v in range(OUT_F // L):
                sl = pl.ds(v * L, L)
                rows_v[ei, sl] = rows_v[ei, sl] * ps

        pltpu.sync_copy(rows_v, hp_sh.at[cdx], add=True)
        return 0

    def _superblock(sb, _):
        pltpu.sync_copy(row_hbm.at[wid, sb], ridx_v)
        pltpu.sync_copy(col_hbm.at[wid, sb], cidx_v)
        lax.fori_loop(0, SB, _chunk, 0)
        return 0

    lax.fori_loop(0, NSB, _superblock, 0)

    plsc.subcore_barrier()

    pltpu.sync_copy(s_v, s_hbm.at[wid])
    pltpu.sync_copy(hp_sh.at[pl.ds(base, ROWS_PER_TILE)],
                    hp_hbm.at[cid, pl.ds(base, ROWS_PER_TILE)])


_sc = pl.kernel(
    _sc_body,
    out_type=[
        jax.ShapeDtypeStruct((NC, N_PAD, OUT_F), jnp.float32),
        jax.ShapeDtypeStruct((NW, 1, N), jnp.float32),
    ],
    mesh=plsc.VectorSubcoreMesh(core_axis_name="c", subcore_axis_name="s"),
    compiler_params=pltpu.CompilerParams(needs_layout_passes=False),
    scratch_types=[
        pltpu.VMEM((L,), jnp.float32),
        pltpu.VMEM((SB, 1, CHUNK_P), jnp.int32),
        pltpu.VMEM((SB, 1, CHUNK_P), jnp.int32),
        pltpu.VMEM((CHUNK_P,), jnp.float32),
        pltpu.VMEM((CHUNK_P,), jnp.float32),
        pltpu.VMEM((1, N), jnp.float32),
        pltpu.VMEM((CHUNK_P,), jnp.float32),
        pltpu.VMEM((CHUNK_P, OUT_F), jnp.float32),
        pltpu.VMEM_SHARED((N,), jnp.float32),
        pltpu.VMEM_SHARED((N,), jnp.float32),
        pltpu.VMEM_SHARED((N_PAD, OUT_F), jnp.float32),
        pltpu.SemaphoreType.DMA,
    ],
)


def _post_body(hp_ref, sp_ref, wh_ref, wh1_ref, wh2_ref, m1_ref, o_ref):
    s = jnp.sum(sp_ref[...], axis=1, keepdims=True)
    wh1 = wh1_ref[...]
    wh2 = wh2_ref[...]
    m1 = m1_ref[...]
    mc = _leaky(wh2 + m1)
    p_self = jnp.exp(_leaky(wh1 + wh2) - mc)
    p_self = jnp.where(s > 0.0, p_self, 0.0)
    denom = s + p_self + jnp.float32(1e-16)
    num = hp_ref[0] + hp_ref[1] + p_self * wh_ref[...]
    hp = num / denom
    o_ref[...] = jnp.where(hp > 0.0, hp, jnp.exp(hp) - 1.0)


_post = pl.pallas_call(
    _post_body,
    out_shape=jax.ShapeDtypeStruct((N, OUT_F), jnp.float32),
)


def kernel(h, adj, W, a):
    a1 = a[:OUT_F]
    a2 = a[OUT_F:]
    wh, wh1, wh2, m1 = _pre(h, W, a1, a2)
    row = adj[0].astype(jnp.int32).reshape(NW, NSB * SB, CHUNK)
    col = adj[1].astype(jnp.int32).reshape(NW, NSB * SB, CHUNK)
    rpad = jnp.zeros((NW, NSB * SB, PAD), jnp.int32)
    cpad = jnp.full((NW, NSB * SB, PAD), N, jnp.int32)
    row = jnp.concatenate([rpad, row], axis=2).reshape(NW, NSB, SB, 1, CHUNK_P)
    col = jnp.concatenate([cpad, col], axis=2).reshape(NW, NSB, SB, 1, CHUNK_P)
    m1vec = jnp.broadcast_to(m1.reshape(1), (L,))
    hp, s_parts = _sc(wh, wh1.reshape(-1), wh2.reshape(-1), m1vec, row, col)
    hp = hp[:, :N, :]
    return _post(hp, s_parts.reshape(NW, N).T, wh, wh1, wh2, m1)


# speedup vs baseline: 5.4068x; 1.0001x over previous
import functools

import jax
import jax.numpy as jnp
from jax import lax
from jax.experimental import pallas as pl
from jax.experimental.pallas import tpu as pltpu
from jax.experimental.pallas import tpu_sc as plsc

IN_F = 128
OUT_F = 128
ALPHA = 0.2
N = 10000
E = 320000

NC = 2
NS = 16
L = 16
NW = NC * NS
E_PER_W = E // NW
CHUNK = 80
PAD = 16
CHUNK_P = CHUNK + PAD
SB = 25
NSB = E_PER_W // (SB * CHUNK)
N_PAD = 10240
ROWS_PER_TILE = N_PAD // NS
GROUPS = CHUNK // L


def _leaky(x):
    return jnp.maximum(x, ALPHA * x)


def _pre_body(h_ref, w_ref, a1_ref, a2_ref, wh_ref, wh1_ref, wh2_ref, m1_ref):
    wh = jnp.dot(h_ref[...], w_ref[...], preferred_element_type=jnp.float32)
    wh_ref[...] = wh
    wh1 = jnp.dot(wh, a1_ref[...], preferred_element_type=jnp.float32)
    wh2 = jnp.dot(wh, a2_ref[...], preferred_element_type=jnp.float32)
    wh1_ref[...] = wh1
    wh2_ref[...] = wh2
    m1_ref[...] = jnp.max(wh1, keepdims=True)


_pre = pl.pallas_call(
    _pre_body,
    out_shape=[
        jax.ShapeDtypeStruct((N, OUT_F), jnp.float32),
        jax.ShapeDtypeStruct((N, 1), jnp.float32),
        jax.ShapeDtypeStruct((N, 1), jnp.float32),
        jax.ShapeDtypeStruct((1, 1), jnp.float32),
    ],
)


def _sc_body(wh_hbm, wh1_hbm, wh2_hbm, m1_hbm, row_hbm, col_hbm,
             hp_hbm, s_hbm,
             m1_v, ridx_v, cidx_v, w1_v, w2_v, s_v, p_v, rows_v,
             wh1_sh, wh2_sh, hp_sh, sem):
    cid = lax.axis_index("c")
    sid = lax.axis_index("s")
    wid = sid * NC + cid

    pltpu.sync_copy(m1_hbm, m1_v)

    @pl.when(sid == 0)
    def _():
        pltpu.sync_copy(wh1_hbm, wh1_sh)
        pltpu.sync_copy(wh2_hbm, wh2_sh)

    zero16 = jnp.zeros((L,), jnp.float32)

    def _zs(i, _):
        s_v[0, pl.ds(i * L, L)] = zero16
        return 0
    lax.fori_loop(0, N // L, _zs, 0)

    def _zr(i, _):
        for v in range(OUT_F // L):
            rows_v[i, pl.ds(v * L, L)] = zero16
        return 0
    lax.fori_loop(0, CHUNK_P, _zr, 0)

    for g in range(PAD // L):
        p_v[pl.ds(g * L, L)] = zero16

    base = sid * ROWS_PER_TILE
    for k in range(ROWS_PER_TILE // CHUNK):
        pltpu.sync_copy(rows_v.at[pl.ds(0, CHUNK)],
                        hp_sh.at[pl.ds(base + k * CHUNK, CHUNK)])

    plsc.subcore_barrier()

    m1s = m1_v[...]

    def _chunk(c, _):
        idx = ridx_v.at[c, 0]
        cdx = cidx_v.at[c, 0]
        rdesc = pltpu.async_copy(wh_hbm.at[idx], rows_v, sem)
        pltpu.sync_copy(wh1_sh.at[idx], w1_v)
        pltpu.sync_copy(wh2_sh.at[cdx], w2_v)

        for g in range(GROUPS):
            sl = pl.ds(PAD + g * L, L)
            w1 = w1_v[sl]
            w2 = w2_v[sl]
            c16 = cidx_v[c, 0, sl]
            e = _leaky(w1 + w2)
            mc = _leaky(w2 + m1s)
            p16 = jnp.exp(e - mc)
            p_v[sl] = p16
            plsc.addupdate_scatter(s_v, [jnp.zeros((L,), jnp.int32), c16], p16)

        rdesc.wait()

        for ei in range(CHUNK_P):
            ps = plsc.load_gather(p_v, [jnp.full((L,), ei, jnp.int32)])
            for v in range(OUT_F // L):
                sl = pl.ds(v * L, L)
                rows_v[ei, sl] = rows_v[ei, sl] * ps

        pltpu.sync_copy(rows_v, hp_sh.at[cdx], add=True)
        return 0

    def _superblock(sb, _):
        pltpu.sync_copy(row_hbm.at[wid, sb], ridx_v)
        pltpu.sync_copy(col_hbm.at[wid, sb], cidx_v)
        lax.fori_loop(0, SB, _chunk, 0)
        return 0

    lax.fori_loop(0, NSB, _superblock, 0)

    plsc.subcore_barrier()

    pltpu.sync_copy(s_v, s_hbm.at[wid])
    pltpu.sync_copy(hp_sh.at[pl.ds(base, ROWS_PER_TILE)],
                    hp_hbm.at[cid, pl.ds(base, ROWS_PER_TILE)])


_sc = pl.kernel(
    _sc_body,
    out_type=[
        jax.ShapeDtypeStruct((NC, N_PAD, OUT_F), jnp.float32),
        jax.ShapeDtypeStruct((NW, 1, N), jnp.float32),
    ],
    mesh=plsc.VectorSubcoreMesh(core_axis_name="c", subcore_axis_name="s"),
    compiler_params=pltpu.CompilerParams(needs_layout_passes=False),
    scratch_types=[
        pltpu.VMEM((L,), jnp.float32),
        pltpu.VMEM((SB, 1, CHUNK_P), jnp.int32),
        pltpu.VMEM((SB, 1, CHUNK_P), jnp.int32),
        pltpu.VMEM((CHUNK_P,), jnp.float32),
        pltpu.VMEM((CHUNK_P,), jnp.float32),
        pltpu.VMEM((1, N), jnp.float32),
        pltpu.VMEM((CHUNK_P,), jnp.float32),
        pltpu.VMEM((CHUNK_P, OUT_F), jnp.float32),
        pltpu.VMEM_SHARED((N,), jnp.float32),
        pltpu.VMEM_SHARED((N,), jnp.float32),
        pltpu.VMEM_SHARED((N_PAD, OUT_F), jnp.float32),
        pltpu.SemaphoreType.DMA,
    ],
)


def _post_body(hp_ref, sp_ref, wh_ref, wh1_ref, wh2_ref, m1_ref, o_ref):
    s = jnp.sum(sp_ref[...], axis=1, keepdims=True)
    wh1 = wh1_ref[...]
    wh2 = wh2_ref[...]
    m1 = m1_ref[...]
    mc = _leaky(wh2 + m1)
    p_self = jnp.exp(_leaky(wh1 + wh2) - mc)
    p_self = jnp.where(s > 0.0, p_self, 0.0)
    denom = s + p_self + jnp.float32(1e-16)
    num = hp_ref[0] + hp_ref[1] + p_self * wh_ref[...]
    hp = num / denom
    o_ref[...] = jnp.where(hp > 0.0, hp, jnp.exp(hp) - 1.0)


_post = pl.pallas_call(
    _post_body,
    out_shape=jax.ShapeDtypeStruct((N, OUT_F), jnp.float32),
)


def kernel(h, adj, W, a):
    a1 = a[:OUT_F]
    a2 = a[OUT_F:]
    wh, wh1, wh2, m1 = _pre(h, W, a1, a2)
    row = adj[0].astype(jnp.int32).reshape(NW, NSB * SB, CHUNK)
    col = adj[1].astype(jnp.int32).reshape(NW, NSB * SB, CHUNK)
    rpad = jnp.zeros((NW, NSB * SB, PAD), jnp.int32)
    cpad = jnp.full((NW, NSB * SB, PAD), N, jnp.int32)
    row = jnp.concatenate([rpad, row], axis=2).reshape(NW, NSB, SB, 1, CHUNK_P)
    col = jnp.concatenate([cpad, col], axis=2).reshape(NW, NSB, SB, 1, CHUNK_P)
    m1vec = jnp.broadcast_to(m1.reshape(1), (L,))
    hp, s_parts = _sc(wh, wh1.reshape(-1), wh2.reshape(-1), m1vec, row, col)
    hp = hp[:, :N, :]
    return _post(hp, s_parts.reshape(NW, N).T, wh, wh1, wh2, m1)


# speedup vs baseline: 6.9283x; 1.2814x over previous
import functools

import jax
import jax.numpy as jnp
from jax import lax
from jax.experimental import pallas as pl
from jax.experimental.pallas import tpu as pltpu
from jax.experimental.pallas import tpu_sc as plsc

IN_F = 128
OUT_F = 128
ALPHA = 0.2
N = 10000
E = 320000

NC = 2
NS = 16
L = 16
NW = NC * NS
E_PER_W = E // NW
CHUNK = 112
PAD = 16
CHUNK_P = CHUNK + PAD
E_PAD_W = 10080
SB = 18
NSB = E_PAD_W // (SB * CHUNK)
N_PAD = 10240
ROWS_PER_TILE = N_PAD // NS
GROUPS = CHUNK // L


def _leaky(x):
    return jnp.maximum(x, ALPHA * x)


def _pre_body(h_ref, w_ref, a1_ref, a2_ref, wh_ref, wh1_ref, wh2_ref, m1_ref):
    wh = jnp.dot(h_ref[...], w_ref[...], preferred_element_type=jnp.float32)
    wh_ref[...] = wh
    wh1 = jnp.dot(wh, a1_ref[...], preferred_element_type=jnp.float32)
    wh2 = jnp.dot(wh, a2_ref[...], preferred_element_type=jnp.float32)
    wh1_ref[...] = wh1
    wh2_ref[...] = wh2
    m1_ref[...] = jnp.max(wh1, keepdims=True)


_pre = pl.pallas_call(
    _pre_body,
    out_shape=[
        jax.ShapeDtypeStruct((N, OUT_F), jnp.float32),
        jax.ShapeDtypeStruct((N, 1), jnp.float32),
        jax.ShapeDtypeStruct((N, 1), jnp.float32),
        jax.ShapeDtypeStruct((1, 1), jnp.float32),
    ],
)


def _sc_body(wh_hbm, wh1_hbm, wh2_hbm, m1_hbm, row_hbm, col_hbm,
             hp_hbm, s_hbm,
             m1_v, ridx_v, cidx_v, w1_v, w2_v, s_v, p_v, rows_v,
             wh1_sh, wh2_sh, hp_sh, sem):
    cid = lax.axis_index("c")
    sid = lax.axis_index("s")
    wid = sid * NC + cid

    pltpu.sync_copy(m1_hbm, m1_v)

    @pl.when(sid == 0)
    def _():
        pltpu.sync_copy(wh1_hbm, wh1_sh)
        pltpu.sync_copy(wh2_hbm, wh2_sh)

    zero16 = jnp.zeros((L,), jnp.float32)

    def _zs(i, _):
        s_v[0, pl.ds(i * L, L)] = zero16
        return 0
    lax.fori_loop(0, N_PAD // L, _zs, 0)

    def _zr(i, _):
        for v in range(OUT_F // L):
            rows_v[i, pl.ds(v * L, L)] = zero16
        return 0
    lax.fori_loop(0, CHUNK_P, _zr, 0)

    for g in range(PAD // L):
        p_v[pl.ds(g * L, L)] = zero16

    base = sid * ROWS_PER_TILE
    for k in range(ROWS_PER_TILE // 80):
        pltpu.sync_copy(rows_v.at[pl.ds(0, 80)],
                        hp_sh.at[pl.ds(base + k * 80, 80)])

    plsc.subcore_barrier()

    m1s = m1_v[...]

    def _chunk(c, _):
        idx = ridx_v.at[c, 0]
        cdx = cidx_v.at[c, 0]
        rdesc = pltpu.async_copy(wh_hbm.at[idx], rows_v, sem)
        pltpu.sync_copy(wh1_sh.at[idx], w1_v)
        pltpu.sync_copy(wh2_sh.at[cdx], w2_v)

        for g in range(GROUPS):
            sl = pl.ds(PAD + g * L, L)
            w1 = w1_v[sl]
            w2 = w2_v[sl]
            c16 = cidx_v[c, 0, sl]
            e = _leaky(w1 + w2)
            mc = _leaky(w2 + m1s)
            p16 = jnp.exp(e - mc)
            p_v[sl] = p16
            plsc.addupdate_scatter(s_v, [jnp.zeros((L,), jnp.int32), c16], p16)

        rdesc.wait()

        for ei in range(CHUNK_P):
            ps = plsc.load_gather(p_v, [jnp.full((L,), ei, jnp.int32)])
            for v in range(OUT_F // L):
                sl = pl.ds(v * L, L)
                rows_v[ei, sl] = rows_v[ei, sl] * ps

        pltpu.sync_copy(rows_v, hp_sh.at[cdx], add=True)
        return 0

    def _superblock(sb, _):
        pltpu.sync_copy(row_hbm.at[wid, sb], ridx_v)
        pltpu.sync_copy(col_hbm.at[wid, sb], cidx_v)
        lax.fori_loop(0, SB, _chunk, 0)
        return 0

    lax.fori_loop(0, NSB, _superblock, 0)

    plsc.subcore_barrier()

    pltpu.sync_copy(s_v, s_hbm.at[wid])
    pltpu.sync_copy(hp_sh.at[pl.ds(base, ROWS_PER_TILE)],
                    hp_hbm.at[cid, pl.ds(base, ROWS_PER_TILE)])


_sc = pl.kernel(
    _sc_body,
    out_type=[
        jax.ShapeDtypeStruct((NC, N_PAD, OUT_F), jnp.float32),
        jax.ShapeDtypeStruct((NW, 1, N_PAD), jnp.float32),
    ],
    mesh=plsc.VectorSubcoreMesh(core_axis_name="c", subcore_axis_name="s"),
    compiler_params=pltpu.CompilerParams(needs_layout_passes=False),
    scratch_types=[
        pltpu.VMEM((L,), jnp.float32),
        pltpu.VMEM((SB, 1, CHUNK_P), jnp.int32),
        pltpu.VMEM((SB, 1, CHUNK_P), jnp.int32),
        pltpu.VMEM((CHUNK_P,), jnp.float32),
        pltpu.VMEM((CHUNK_P,), jnp.float32),
        pltpu.VMEM((1, N_PAD), jnp.float32),
        pltpu.VMEM((CHUNK_P,), jnp.float32),
        pltpu.VMEM((CHUNK_P, OUT_F), jnp.float32),
        pltpu.VMEM_SHARED((N,), jnp.float32),
        pltpu.VMEM_SHARED((N_PAD,), jnp.float32),
        pltpu.VMEM_SHARED((N_PAD, OUT_F), jnp.float32),
        pltpu.SemaphoreType.DMA,
    ],
)


def _post_body(hp_ref, sp_ref, wh_ref, wh1_ref, wh2_ref, m1_ref, o_ref):
    s = jnp.sum(sp_ref[...], axis=1, keepdims=True)
    wh1 = wh1_ref[...]
    wh2 = wh2_ref[...]
    m1 = m1_ref[...]
    mc = _leaky(wh2 + m1)
    p_self = jnp.exp(_leaky(wh1 + wh2) - mc)
    p_self = jnp.where(s > 0.0, p_self, 0.0)
    denom = s + p_self + jnp.float32(1e-16)
    num = hp_ref[0] + hp_ref[1] + p_self * wh_ref[...]
    hp = num / denom
    o_ref[...] = jnp.where(hp > 0.0, hp, jnp.exp(hp) - 1.0)


_post = pl.pallas_call(
    _post_body,
    out_shape=jax.ShapeDtypeStruct((N, OUT_F), jnp.float32),
)


def kernel(h, adj, W, a):
    a1 = a[:OUT_F]
    a2 = a[OUT_F:]
    wh, wh1, wh2, m1 = _pre(h, W, a1, a2)
    nfake = E_PAD_W - E_PER_W
    row = adj[0].astype(jnp.int32).reshape(NW, E_PER_W)
    col = adj[1].astype(jnp.int32).reshape(NW, E_PER_W)
    row = jnp.concatenate([row, jnp.zeros((NW, nfake), jnp.int32)], axis=1)
    col = jnp.concatenate([col, jnp.full((NW, nfake), N, jnp.int32)], axis=1)
    row = row.reshape(NW, NSB * SB, CHUNK)
    col = col.reshape(NW, NSB * SB, CHUNK)
    rpad = jnp.zeros((NW, NSB * SB, PAD), jnp.int32)
    cpad = jnp.full((NW, NSB * SB, PAD), N, jnp.int32)
    row = jnp.concatenate([rpad, row], axis=2).reshape(NW, NSB, SB, 1, CHUNK_P)
    col = jnp.concatenate([cpad, col], axis=2).reshape(NW, NSB, SB, 1, CHUNK_P)
    m1vec = jnp.broadcast_to(m1.reshape(1), (L,))
    wh2p = jnp.pad(wh2.reshape(-1), (0, N_PAD - N))
    hp, s_parts = _sc(wh, wh1.reshape(-1), wh2p, m1vec, row, col)
    hp = hp[:, :N, :]
    sp = s_parts.reshape(NW, N_PAD)[:, :N]
    return _post(hp, sp.T, wh, wh1, wh2, m1)


# speedup vs baseline: 6.9309x; 1.0004x over previous
import functools

import jax
import jax.numpy as jnp
from jax import lax
from jax.experimental import pallas as pl
from jax.experimental.pallas import tpu as pltpu
from jax.experimental.pallas import tpu_sc as plsc

IN_F = 128
OUT_F = 128
ALPHA = 0.2
N = 10000
E = 320000

NC = 2
NS = 16
L = 16
NW = NC * NS
E_PER_W = E // NW
CHUNK = 112
PAD = 16
CHUNK_P = CHUNK + PAD
E_PAD_W = 10080
SB = 9
NSB = E_PAD_W // (SB * CHUNK)
N_PAD = 10240
ROWS_PER_TILE = N_PAD // NS
GROUPS = CHUNK // L


def _leaky(x):
    return jnp.maximum(x, ALPHA * x)


def _pre_body(h_ref, w_ref, a1_ref, a2_ref, wh_ref, wh1_ref, wh2_ref, m1_ref):
    wh = jnp.dot(h_ref[...], w_ref[...], preferred_element_type=jnp.float32)
    wh_ref[...] = wh
    wh1 = jnp.dot(wh, a1_ref[...], preferred_element_type=jnp.float32)
    wh2 = jnp.dot(wh, a2_ref[...], preferred_element_type=jnp.float32)
    wh1_ref[...] = wh1
    wh2_ref[...] = wh2
    m1_ref[...] = jnp.max(wh1, keepdims=True)


_pre = pl.pallas_call(
    _pre_body,
    out_shape=[
        jax.ShapeDtypeStruct((N, OUT_F), jnp.float32),
        jax.ShapeDtypeStruct((N, 1), jnp.float32),
        jax.ShapeDtypeStruct((N, 1), jnp.float32),
        jax.ShapeDtypeStruct((1, 1), jnp.float32),
    ],
)


def _sc_body(wh_hbm, wh1_hbm, wh2_hbm, m1_hbm, row_hbm, col_hbm,
             hp_hbm, s_hbm,
             m1_v, ridx_v, cidx_v, w1_v, w2_v, s_v, p_v, rows_v,
             wh1_sh, wh2_sh, hp_sh, sem, sem2):
    cid = lax.axis_index("c")
    sid = lax.axis_index("s")
    wid = sid * NC + cid

    pltpu.sync_copy(m1_hbm, m1_v)

    @pl.when(sid == 0)
    def _():
        pltpu.sync_copy(wh1_hbm, wh1_sh)
        pltpu.sync_copy(wh2_hbm, wh2_sh)

    zero16 = jnp.zeros((L,), jnp.float32)

    def _zs(i, _):
        s_v[0, pl.ds(i * L, L)] = zero16
        return 0
    lax.fori_loop(0, N_PAD // L, _zs, 0)

    def _zr(i, _):
        for v in range(OUT_F // L):
            rows_v[i, pl.ds(v * L, L)] = zero16
        return 0
    lax.fori_loop(0, CHUNK_P, _zr, 0)

    for g in range(PAD // L):
        p_v[pl.ds(g * L, L)] = zero16

    base = sid * ROWS_PER_TILE
    for k in range(ROWS_PER_TILE // 80):
        pltpu.sync_copy(rows_v.at[pl.ds(0, 80)],
                        hp_sh.at[pl.ds(base + k * 80, 80)])

    plsc.subcore_barrier()

    m1s = m1_v[...]

    def _chunk(c, _):
        idx = ridx_v.at[c, 0]
        cdx = cidx_v.at[c, 0]
        rdesc = pltpu.async_copy(wh_hbm.at[idx], rows_v, sem)
        d1 = pltpu.async_copy(wh1_sh.at[idx], w1_v, sem2)
        d2 = pltpu.async_copy(wh2_sh.at[cdx], w2_v, sem2)
        d1.wait()
        d2.wait()

        for g in range(GROUPS):
            sl = pl.ds(PAD + g * L, L)
            w1 = w1_v[sl]
            w2 = w2_v[sl]
            c16 = cidx_v[c, 0, sl]
            e = _leaky(w1 + w2)
            mc = _leaky(w2 + m1s)
            p16 = jnp.exp(e - mc)
            p_v[sl] = p16
            plsc.addupdate_scatter(s_v, [jnp.zeros((L,), jnp.int32), c16], p16)

        rdesc.wait()

        for ei in range(CHUNK_P):
            ps = plsc.load_gather(p_v, [jnp.full((L,), ei, jnp.int32)])
            for v in range(OUT_F // L):
                sl = pl.ds(v * L, L)
                rows_v[ei, sl] = rows_v[ei, sl] * ps

        pltpu.sync_copy(rows_v, hp_sh.at[cdx], add=True)
        return 0

    def _superblock(sb, _):
        pltpu.sync_copy(row_hbm.at[wid, sb], ridx_v)
        pltpu.sync_copy(col_hbm.at[wid, sb], cidx_v)
        lax.fori_loop(0, SB, _chunk, 0)
        return 0

    lax.fori_loop(0, NSB, _superblock, 0)

    plsc.subcore_barrier()

    pltpu.sync_copy(s_v, s_hbm.at[wid])
    pltpu.sync_copy(hp_sh.at[pl.ds(base, ROWS_PER_TILE)],
                    hp_hbm.at[cid, pl.ds(base, ROWS_PER_TILE)])


_sc = pl.kernel(
    _sc_body,
    out_type=[
        jax.ShapeDtypeStruct((NC, N_PAD, OUT_F), jnp.float32),
        jax.ShapeDtypeStruct((NW, 1, N_PAD), jnp.float32),
    ],
    mesh=plsc.VectorSubcoreMesh(core_axis_name="c", subcore_axis_name="s"),
    compiler_params=pltpu.CompilerParams(needs_layout_passes=False),
    scratch_types=[
        pltpu.VMEM((L,), jnp.float32),
        pltpu.VMEM((SB, 1, CHUNK_P), jnp.int32),
        pltpu.VMEM((SB, 1, CHUNK_P), jnp.int32),
        pltpu.VMEM((CHUNK_P,), jnp.float32),
        pltpu.VMEM((CHUNK_P,), jnp.float32),
        pltpu.VMEM((1, N_PAD), jnp.float32),
        pltpu.VMEM((CHUNK_P,), jnp.float32),
        pltpu.VMEM((CHUNK_P, OUT_F), jnp.float32),
        pltpu.VMEM_SHARED((N,), jnp.float32),
        pltpu.VMEM_SHARED((N_PAD,), jnp.float32),
        pltpu.VMEM_SHARED((N_PAD, OUT_F), jnp.float32),
        pltpu.SemaphoreType.DMA,
        pltpu.SemaphoreType.DMA,
    ],
)


def _post_body(hp_ref, sp_ref, wh_ref, wh1_ref, wh2_ref, m1_ref, o_ref):
    s = jnp.sum(sp_ref[...], axis=1, keepdims=True)
    wh1 = wh1_ref[...]
    wh2 = wh2_ref[...]
    m1 = m1_ref[...]
    mc = _leaky(wh2 + m1)
    p_self = jnp.exp(_leaky(wh1 + wh2) - mc)
    p_self = jnp.where(s > 0.0, p_self, 0.0)
    denom = s + p_self + jnp.float32(1e-16)
    num = hp_ref[0] + hp_ref[1] + p_self * wh_ref[...]
    hp = num / denom
    o_ref[...] = jnp.where(hp > 0.0, hp, jnp.exp(hp) - 1.0)


_post = pl.pallas_call(
    _post_body,
    out_shape=jax.ShapeDtypeStruct((N, OUT_F), jnp.float32),
)


def kernel(h, adj, W, a):
    a1 = a[:OUT_F]
    a2 = a[OUT_F:]
    wh, wh1, wh2, m1 = _pre(h, W, a1, a2)
    nfake = E_PAD_W - E_PER_W
    row = adj[0].astype(jnp.int32).reshape(NW, E_PER_W)
    col = adj[1].astype(jnp.int32).reshape(NW, E_PER_W)
    row = jnp.concatenate([row, jnp.zeros((NW, nfake), jnp.int32)], axis=1)
    col = jnp.concatenate([col, jnp.full((NW, nfake), N, jnp.int32)], axis=1)
    row = row.reshape(NW, NSB * SB, CHUNK)
    col = col.reshape(NW, NSB * SB, CHUNK)
    rpad = jnp.zeros((NW, NSB * SB, PAD), jnp.int32)
    cpad = jnp.full((NW, NSB * SB, PAD), N, jnp.int32)
    row = jnp.concatenate([rpad, row], axis=2).reshape(NW, NSB, SB, 1, CHUNK_P)
    col = jnp.concatenate([cpad, col], axis=2).reshape(NW, NSB, SB, 1, CHUNK_P)
    m1vec = jnp.broadcast_to(m1.reshape(1), (L,))
    wh2p = jnp.pad(wh2.reshape(-1), (0, N_PAD - N))
    hp, s_parts = _sc(wh, wh1.reshape(-1), wh2p, m1vec, row, col)
    hp = hp[:, :N, :]
    sp = s_parts.reshape(NW, N_PAD)[:, :N]
    return _post(hp, sp.T, wh, wh1, wh2, m1)
